# Initial kernel scaffold; baseline (speedup 1.0000x reference)
import functools

import jax
import jax.numpy as jnp
from jax import lax
from jax.experimental import pallas as pl
from jax.experimental.pallas import tpu as pltpu
from jax.experimental.pallas import tpu_sc as plsc

N = 10000
E = 320000
IN = 128
H = 128
C = 40
CP = 64
R = 8
NB = 30

NC = 2
NS = 16
L = 16

NBLK = 10
BN = N // NBLK

K = 80
EPT_ALL = E // NS
EPT_HALF = E // (NC * NS)


def _prep1_body(x_ref, comp_ref, bases_ref, hall_ref):
    bas = bases_ref[...].reshape(NB, IN * H)
    w = jnp.dot(comp_ref[0], bas, preferred_element_type=jnp.float32)
    w = w.reshape(IN, H)
    hall_ref[0] = jnp.dot(x_ref[...], w, preferred_element_type=jnp.float32)


def _tc_prep1(x, comp1, bases1):
    comp3 = comp1.reshape(R, 1, NB)
    return pl.pallas_call(
        _prep1_body,
        grid=(R, NBLK),
        in_specs=[
            pl.BlockSpec((BN, IN), lambda r, b: (b, 0)),
            pl.BlockSpec((1, 1, NB), lambda r, b: (r, 0, 0)),
            pl.BlockSpec((NB, IN, H), lambda r, b: (0, 0, 0)),
        ],
        out_specs=pl.BlockSpec((1, BN, H), lambda r, b: (r, b, 0)),
        out_shape=jax.ShapeDtypeStruct((R, N, H), jnp.float32),
    )(x, comp3, bases1)


def _mid_body(x_ref, a_ref, root1_ref, bias1_ref, comp2_ref, bases2_ref,
              root2_ref, bias2_ref, hall2_ref, self2_ref):
    h = jnp.dot(x_ref[...], root1_ref[...], preferred_element_type=jnp.float32)
    h = h + bias1_ref[...] + a_ref[0] + a_ref[1]
    h = jnp.maximum(h, 0.0)
    bas = bases2_ref[...].reshape(NB, H * CP)
    w = jnp.dot(comp2_ref[0], bas, preferred_element_type=jnp.float32)
    w = w.reshape(H, CP)
    hall2_ref[0] = jnp.dot(h, w, preferred_element_type=jnp.float32)
    self2_ref[...] = (
        jnp.dot(h, root2_ref[...], preferred_element_type=jnp.float32)
        + bias2_ref[...]
    )


def _tc_mid(x, a_parts, root1, bias1, comp2, bases2p, root2p, bias2p):
    comp3 = comp2.reshape(R, 1, NB)
    return pl.pallas_call(
        _mid_body,
        grid=(R, NBLK),
        in_specs=[
            pl.BlockSpec((BN, IN), lambda r, b: (b, 0)),
            pl.BlockSpec((NC, BN, H), lambda r, b: (0, b, 0)),
            pl.BlockSpec((IN, H), lambda r, b: (0, 0)),
            pl.BlockSpec((1, H), lambda r, b: (0, 0)),
            pl.BlockSpec((1, 1, NB), lambda r, b: (r, 0, 0)),
            pl.BlockSpec((NB, H, CP), lambda r, b: (0, 0, 0)),
            pl.BlockSpec((H, CP), lambda r, b: (0, 0)),
            pl.BlockSpec((1, CP), lambda r, b: (0, 0)),
        ],
        out_specs=[
            pl.BlockSpec((1, BN, CP), lambda r, b: (r, b, 0)),
            pl.BlockSpec((BN, CP), lambda r, b: (b, 0)),
        ],
        out_shape=[
            jax.ShapeDtypeStruct((R, N, CP), jnp.float32),
            jax.ShapeDtypeStruct((N, CP), jnp.float32),
        ],
    )(x, a_parts, root1, bias1.reshape(1, H), comp3, bases2p, root2p, bias2p)


def _final_body(self2_ref, b_ref, out_ref):
    z = self2_ref[...] + b_ref[0] + b_ref[1]
    col = lax.broadcasted_iota(jnp.int32, (BN, CP), 1)
    valid = col < C
    zm = jnp.where(valid, z, -jnp.inf)
    m = jnp.max(zm, axis=1, keepdims=True)
    ex = jnp.where(valid, jnp.exp(z - m), 0.0)
    lse = jnp.log(jnp.sum(ex, axis=1, keepdims=True))
    out_ref[...] = z - m - lse


def _tc_final(self2, b_parts):
    return pl.pallas_call(
        _final_body,
        grid=(NBLK,),
        in_specs=[
            pl.BlockSpec((BN, CP), lambda b: (b, 0)),
            pl.BlockSpec((NC, BN, CP), lambda b: (0, b, 0)),
        ],
        out_specs=pl.BlockSpec((BN, CP), lambda b: (b, 0)),
        out_shape=jax.ShapeDtypeStruct((N, CP), jnp.float32),
    )(self2, b_parts)


_MESH = dict(core_axis_name="c", subcore_axis_name="s")
NPT = N // NS


def _edge_pass(hall, src_hbm, dst_hbm, t_hbm, inv_sh, a_sh,
               srcb, dstb, tb, gidxb, invrows, scb, rows, width, c, s):
    iota = lax.iota(jnp.int32, L)
    nslices = width // L
    base0 = c * (E // NC) + s * EPT_HALF

    @pl.loop(0, EPT_HALF // K)
    def _chunk(ci):
        base = base0 + ci * K
        pltpu.sync_copy(src_hbm.at[pl.ds(base, K)], srcb)
        pltpu.sync_copy(dst_hbm.at[pl.ds(base, K)], dstb)
        pltpu.sync_copy(t_hbm.at[pl.ds(base, K)], tb)
        for g in range(K // L):
            tv = tb[pl.ds(g * L, L)]
            sv = srcb[pl.ds(g * L, L)]
            gidxb[pl.ds(g * L, L)] = tv * N + sv
        pltpu.sync_copy(hall.at[gidxb], rows)
        pltpu.sync_copy(inv_sh.at[dstb], invrows)
        for g in range(K // L):
            ridx = iota + g * L
            tv = tb[pl.ds(g * L, L)]
            scb[pl.ds(g * L, L)] = plsc.load_gather(invrows, [ridx, tv])

        @pl.loop(0, K)
        def _scale(k):
            spl = plsc.load_gather(scb, [lax.broadcast(k, (L,))])
            for j in range(nslices):
                sl = rows[k, pl.ds(j * L, L)]
                rows[k, pl.ds(j * L, L)] = sl * spl

        pltpu.sync_copy(rows, a_sh.at[dstb], add=True)


def _msg1_kernel(hall, src_hbm, dst_hbm, t_hbm, z128, z16,
                 out_parts, inv_out,
                 a_sh, inv_sh, srcb, dstb, tb, gidxb, invrows, scb,
                 rows, oh, cntb):
    c = lax.axis_index("c")
    s = lax.axis_index("s")
    iota = lax.iota(jnp.int32, L)
    ones = jnp.ones((L,), jnp.float32)
    zeros = jnp.zeros((L,), jnp.float32)

    sl = pl.ds(s * NPT, NPT)
    pltpu.sync_copy(z128.at[sl], a_sh.at[sl])
    pltpu.sync_copy(z16.at[sl], inv_sh.at[sl])

    @pl.loop(0, K)
    def _zoh(i):
        oh[i, :] = zeros

    plsc.subcore_barrier()

    @pl.loop(0, EPT_ALL // K)
    def _cchunk(ci):
        base = s * EPT_ALL + ci * K
        pltpu.sync_copy(dst_hbm.at[pl.ds(base, K)], dstb)
        pltpu.sync_copy(t_hbm.at[pl.ds(base, K)], tb)
        for g in range(K // L):
            ridx = iota + g * L
            tv = tb[pl.ds(g * L, L)]
            plsc.store_scatter(oh, [ridx, tv], ones)
        pltpu.sync_copy(oh, inv_sh.at[dstb], add=True)
        for g in range(K // L):
            ridx = iota + g * L
            tv = tb[pl.ds(g * L, L)]
            plsc.store_scatter(oh, [ridx, tv], zeros)

    plsc.subcore_barrier()

    pltpu.sync_copy(inv_sh.at[sl], cntb)

    @pl.loop(0, NPT)
    def _inv(i):
        cntb[i, :] = 1.0 / jnp.maximum(cntb[i, :], 1.0)

    pltpu.sync_copy(cntb, inv_sh.at[sl])

    @pl.when(c == 0)
    def _():
        pltpu.sync_copy(cntb, inv_out.at[sl])

    plsc.subcore_barrier()

    _edge_pass(hall, src_hbm, dst_hbm, t_hbm, inv_sh, a_sh,
               srcb, dstb, tb, gidxb, invrows, scb, rows, H, c, s)

    plsc.subcore_barrier()
    pltpu.sync_copy(a_sh.at[sl], out_parts.at[c, sl])


def _msg2_kernel(hall, src_hbm, dst_hbm, t_hbm, z64, inv_hbm,
                 out_parts,
                 a_sh, inv_sh, srcb, dstb, tb, gidxb, invrows, scb, rows):
    c = lax.axis_index("c")
    s = lax.axis_index("s")
    sl = pl.ds(s * NPT, NPT)
    pltpu.sync_copy(z64.at[sl], a_sh.at[sl])
    pltpu.sync_copy(inv_hbm.at[sl], inv_sh.at[sl])
    plsc.subcore_barrier()
    _edge_pass(hall, src_hbm, dst_hbm, t_hbm, inv_sh, a_sh,
               srcb, dstb, tb, gidxb, invrows, scb, rows, CP, c, s)
    plsc.subcore_barrier()
    pltpu.sync_copy(a_sh.at[sl], out_parts.at[c, sl])


def _sc_msg1(hall1, src, dst, et, z128, z16):
    f = pl.kernel(
        _msg1_kernel,
        out_type=[
            jax.ShapeDtypeStruct((NC, N, H), jnp.float32),
            jax.ShapeDtypeStruct((N, L), jnp.float32),
        ],
        mesh=plsc.VectorSubcoreMesh(**_MESH),
        scratch_types=[
            pltpu.VMEM_SHARED((N, H), jnp.float32),
            pltpu.VMEM_SHARED((N, L), jnp.float32),
            pltpu.VMEM((K,), jnp.int32),
            pltpu.VMEM((K,), jnp.int32),
            pltpu.VMEM((K,), jnp.int32),
            pltpu.VMEM((K,), jnp.int32),
            pltpu.VMEM((K, L), jnp.float32),
            pltpu.VMEM((K,), jnp.float32),
            pltpu.VMEM((K, H), jnp.float32),
            pltpu.VMEM((K, L), jnp.float32),
            pltpu.VMEM((NPT, L), jnp.float32),
        ],
    )
    return f(hall1, src, dst, et, z128, z16)


def _sc_msg2(hall2, src, dst, et, z64, inv):
    f = pl.kernel(
        _msg2_kernel,
        out_type=jax.ShapeDtypeStruct((NC, N, CP), jnp.float32),
        mesh=plsc.VectorSubcoreMesh(**_MESH),
        scratch_types=[
            pltpu.VMEM_SHARED((N, CP), jnp.float32),
            pltpu.VMEM_SHARED((N, L), jnp.float32),
            pltpu.VMEM((K,), jnp.int32),
            pltpu.VMEM((K,), jnp.int32),
            pltpu.VMEM((K,), jnp.int32),
            pltpu.VMEM((K,), jnp.int32),
            pltpu.VMEM((K, L), jnp.float32),
            pltpu.VMEM((K,), jnp.float32),
            pltpu.VMEM((K, CP), jnp.float32),
        ],
    )
    return f(hall2, src, dst, et, z64, inv)


def kernel(x, edge_index, edge_type, comp1, bases1, root1, bias1,
           comp2, bases2, root2, bias2):
    src = edge_index[0]
    dst = edge_index[1]
    et = edge_type

    z128 = jnp.zeros((N, H), jnp.float32)
    z64 = jnp.zeros((N, CP), jnp.float32)
    z16 = jnp.zeros((N, L), jnp.float32)

    bases2p = jnp.pad(bases2, ((0, 0), (0, 0), (0, CP - C)))
    root2p = jnp.pad(root2, ((0, 0), (0, CP - C)))
    bias2p = jnp.pad(bias2, (0, CP - C)).reshape(1, CP)

    hall1 = _tc_prep1(x, comp1, bases1).reshape(R * N, H)
    a_parts, inv = _sc_msg1(hall1, src, dst, et, z128, z16)
    hall2, self2 = _tc_mid(x, a_parts, root1, bias1, comp2, bases2p,
                           root2p, bias2p)
    b_parts = _sc_msg2(hall2.reshape(R * N, CP), src, dst, et, z64, inv)
    out = _tc_final(self2, b_parts)
    return out[:, :C]


# speedup vs baseline: 18.2993x; 18.2993x over previous
import functools

import jax
import jax.numpy as jnp
from jax import lax
from jax.experimental import pallas as pl
from jax.experimental.pallas import tpu as pltpu
from jax.experimental.pallas import tpu_sc as plsc

N = 10000
NP = 10240
E = 320000
IN = 128
H = 128
C = 40
CP = 128
R = 8
NB = 30

NC = 2
NS = 16
L = 16

NBLK = 10
BN = NP // NBLK

K = 80
EPT_ALL = E // NS
EPT_HALF = E // (NC * NS)


def _prep1_body(x_ref, comp_ref, bases_ref, hall_ref):
    bas = bases_ref[...].reshape(NB, IN * H)
    w = jnp.dot(comp_ref[0], bas, preferred_element_type=jnp.float32)
    w = w.reshape(IN, H)
    hall_ref[0] = jnp.dot(x_ref[...], w, preferred_element_type=jnp.float32)


def _tc_prep1(x, comp1, bases1):
    comp3 = comp1.reshape(R, 1, NB)
    return pl.pallas_call(
        _prep1_body,
        grid=(NBLK, R),
        in_specs=[
            pl.BlockSpec((BN, IN), lambda b, r: (b, 0)),
            pl.BlockSpec((1, 1, NB), lambda b, r: (r, 0, 0)),
            pl.BlockSpec((NB, IN, H), lambda b, r: (0, 0, 0)),
        ],
        out_specs=pl.BlockSpec((1, BN, H), lambda b, r: (r, b, 0)),
        out_shape=jax.ShapeDtypeStruct((R, NP, H), jnp.float32),
    )(x, comp3, bases1)


def _mid_body(x_ref, a_ref, root1_ref, bias1_ref, comp2_ref, bases2_ref,
              root2_ref, bias2_ref, hall2_ref, self2_ref):
    h = jnp.dot(x_ref[...], root1_ref[...], preferred_element_type=jnp.float32)
    h = h + bias1_ref[...] + a_ref[0] + a_ref[1]
    h = jnp.maximum(h, 0.0)
    bas = bases2_ref[...].reshape(NB, H * CP)
    w = jnp.dot(comp2_ref[0], bas, preferred_element_type=jnp.float32)
    w = w.reshape(H, CP)
    hall2_ref[0] = jnp.dot(h, w, preferred_element_type=jnp.float32)
    self2_ref[...] = (
        jnp.dot(h, root2_ref[...], preferred_element_type=jnp.float32)
        + bias2_ref[...]
    )


def _tc_mid(x, a_parts, root1, bias1, comp2, bases2p, root2p, bias2p):
    comp3 = comp2.reshape(R, 1, NB)
    return pl.pallas_call(
        _mid_body,
        grid=(NBLK, R),
        in_specs=[
            pl.BlockSpec((BN, IN), lambda b, r: (b, 0)),
            pl.BlockSpec((NC, BN, H), lambda b, r: (0, b, 0)),
            pl.BlockSpec((IN, H), lambda b, r: (0, 0)),
            pl.BlockSpec((1, H), lambda b, r: (0, 0)),
            pl.BlockSpec((1, 1, NB), lambda b, r: (r, 0, 0)),
            pl.BlockSpec((NB, H, CP), lambda b, r: (0, 0, 0)),
            pl.BlockSpec((H, CP), lambda b, r: (0, 0)),
            pl.BlockSpec((1, CP), lambda b, r: (0, 0)),
        ],
        out_specs=[
            pl.BlockSpec((1, BN, CP), lambda b, r: (r, b, 0)),
            pl.BlockSpec((BN, CP), lambda b, r: (b, 0)),
        ],
        out_shape=[
            jax.ShapeDtypeStruct((R, NP, CP), jnp.float32),
            jax.ShapeDtypeStruct((NP, CP), jnp.float32),
        ],
    )(x, a_parts, root1, bias1.reshape(1, H), comp3, bases2p, root2p, bias2p)


def _final_body(self2_ref, b_ref, out_ref):
    z = self2_ref[...] + b_ref[0] + b_ref[1]
    col = lax.broadcasted_iota(jnp.int32, (BN, CP), 1)
    valid = col < C
    zm = jnp.where(valid, z, -jnp.inf)
    m = jnp.max(zm, axis=1, keepdims=True)
    ex = jnp.where(valid, jnp.exp(z - m), 0.0)
    lse = jnp.log(jnp.sum(ex, axis=1, keepdims=True))
    out_ref[...] = z - m - lse


def _tc_final(self2, b_parts):
    return pl.pallas_call(
        _final_body,
        grid=(NBLK,),
        in_specs=[
            pl.BlockSpec((BN, CP), lambda b: (b, 0)),
            pl.BlockSpec((NC, BN, CP), lambda b: (0, b, 0)),
        ],
        out_specs=pl.BlockSpec((BN, CP), lambda b: (b, 0)),
        out_shape=jax.ShapeDtypeStruct((NP, CP), jnp.float32),
    )(self2, b_parts)


_MESH = dict(core_axis_name="c", subcore_axis_name="s")
NPT = NP // NS


def _edge_pass(hall, src_hbm, dst_hbm, t_hbm, inv_sh, a_sh,
               srcb, dstb, tb, gidxb, scb, rows, width, c, s):
    nslices = width // L
    base0 = c * (E // NC) + s * EPT_HALF

    @pl.loop(0, EPT_HALF // K)
    def _chunk(ci):
        base = base0 + ci * K
        pltpu.sync_copy(src_hbm.at[pl.ds(base, K)], srcb)
        pltpu.sync_copy(dst_hbm.at[pl.ds(base, K)], dstb)
        pltpu.sync_copy(t_hbm.at[pl.ds(base, K)], tb)
        for g in range(K // L):
            tv = tb[pl.ds(g * L, L)]
            sv = srcb[pl.ds(g * L, L)]
            gidxb[pl.ds(g * L, L)] = tv * NP + sv
        pltpu.sync_copy(hall.at[gidxb], rows)
        for g in range(K // L):
            tv = tb[pl.ds(g * L, L)]
            dv = dstb[pl.ds(g * L, L)]
            gidxb[pl.ds(g * L, L)] = dv * L + tv
        pltpu.sync_copy(inv_sh.at[gidxb], scb.at[pl.ds(0, K)])

        @pl.loop(0, K)
        def _scale(k):
            v = scb[pl.ds(k, L)]
            spl = lax.broadcast(v[0], (L,))
            for j in range(nslices):
                sl = rows[k, pl.ds(j * L, L)]
                rows[k, pl.ds(j * L, L)] = sl * spl

        pltpu.sync_copy(rows, a_sh.at[dstb], add=True)


def _msg1_kernel(hall, src_hbm, dst_hbm, t_hbm, z128, z16,
                 out_parts, inv_out,
                 a_sh, inv_sh, srcb, dstb, tb, gidxb, scb,
                 rows, onesb, cntb):
    c = lax.axis_index("c")
    s = lax.axis_index("s")

    sl = pl.ds(s * NPT, NPT)
    slf = pl.ds(s * NPT * L, NPT * L)
    pltpu.sync_copy(z128.at[sl], a_sh.at[sl])
    pltpu.sync_copy(z16.at[slf], inv_sh.at[slf])

    @pl.loop(0, K // L)
    def _ones(g):
        onesb[pl.ds(g * L, L)] = jnp.ones((L,), jnp.float32)

    plsc.subcore_barrier()

    @pl.loop(0, EPT_ALL // K)
    def _cchunk(ci):
        base = s * EPT_ALL + ci * K
        pltpu.sync_copy(dst_hbm.at[pl.ds(base, K)], dstb)
        pltpu.sync_copy(t_hbm.at[pl.ds(base, K)], tb)
        for g in range(K // L):
            tv = tb[pl.ds(g * L, L)]
            dv = dstb[pl.ds(g * L, L)]
            gidxb[pl.ds(g * L, L)] = dv * L + tv
        pltpu.sync_copy(onesb, inv_sh.at[gidxb], add=True)

    plsc.subcore_barrier()

    pltpu.sync_copy(inv_sh.at[slf], cntb)

    @pl.loop(0, NPT * L // L)
    def _inv(i):
        v = cntb[pl.ds(i * L, L)]
        cntb[pl.ds(i * L, L)] = 1.0 / jnp.maximum(v, 1.0)

    pltpu.sync_copy(cntb, inv_sh.at[slf])

    @pl.when(c == 0)
    def _():
        pltpu.sync_copy(cntb, inv_out.at[slf])

    plsc.subcore_barrier()

    _edge_pass(hall, src_hbm, dst_hbm, t_hbm, inv_sh, a_sh,
               srcb, dstb, tb, gidxb, scb, rows, H, c, s)

    plsc.subcore_barrier()
    pltpu.sync_copy(a_sh.at[sl], out_parts.at[c, sl])


def _msg2_kernel(hall, src_hbm, dst_hbm, t_hbm, z128, inv_hbm,
                 out_parts,
                 a_sh, inv_sh, srcb, dstb, tb, gidxb, scb, rows):
    c = lax.axis_index("c")
    s = lax.axis_index("s")
    sl = pl.ds(s * NPT, NPT)
    slf = pl.ds(s * NPT * L, NPT * L)
    pltpu.sync_copy(z128.at[sl], a_sh.at[sl])
    pltpu.sync_copy(inv_hbm.at[slf], inv_sh.at[slf])
    plsc.subcore_barrier()
    _edge_pass(hall, src_hbm, dst_hbm, t_hbm, inv_sh, a_sh,
               srcb, dstb, tb, gidxb, scb, rows, CP, c, s)
    plsc.subcore_barrier()
    pltpu.sync_copy(a_sh.at[sl], out_parts.at[c, sl])


def _sc_msg1(hall1, src, dst, et, z128, z16):
    f = pl.kernel(
        _msg1_kernel,
        out_type=[
            jax.ShapeDtypeStruct((NC, NP, H), jnp.float32),
            jax.ShapeDtypeStruct((NP * L,), jnp.float32),
        ],
        mesh=plsc.VectorSubcoreMesh(**_MESH),
        scratch_types=[
            pltpu.VMEM_SHARED((NP, H), jnp.float32),
            pltpu.VMEM_SHARED((NP * L,), jnp.float32),
            pltpu.VMEM((K,), jnp.int32),
            pltpu.VMEM((K,), jnp.int32),
            pltpu.VMEM((K,), jnp.int32),
            pltpu.VMEM((K,), jnp.int32),
            pltpu.VMEM((K + L,), jnp.float32),
            pltpu.VMEM((K, H), jnp.float32),
            pltpu.VMEM((K,), jnp.float32),
            pltpu.VMEM((NPT * L,), jnp.float32),
        ],
    )
    return f(hall1, src, dst, et, z128, z16)


def _sc_msg2(hall2, src, dst, et, z128, inv):
    f = pl.kernel(
        _msg2_kernel,
        out_type=jax.ShapeDtypeStruct((NC, NP, CP), jnp.float32),
        mesh=plsc.VectorSubcoreMesh(**_MESH),
        scratch_types=[
            pltpu.VMEM_SHARED((NP, CP), jnp.float32),
            pltpu.VMEM_SHARED((NP * L,), jnp.float32),
            pltpu.VMEM((K,), jnp.int32),
            pltpu.VMEM((K,), jnp.int32),
            pltpu.VMEM((K,), jnp.int32),
            pltpu.VMEM((K,), jnp.int32),
            pltpu.VMEM((K + L,), jnp.float32),
            pltpu.VMEM((K, CP), jnp.float32),
        ],
    )
    return f(hall2, src, dst, et, z128, inv)


def kernel(x, edge_index, edge_type, comp1, bases1, root1, bias1,
           comp2, bases2, root2, bias2):
    src = edge_index[0]
    dst = edge_index[1]
    et = edge_type

    xp = jnp.pad(x, ((0, NP - N), (0, 0)))
    z128 = jnp.zeros((NP, H), jnp.float32)
    z16 = jnp.zeros((NP * L,), jnp.float32)

    bases2p = jnp.pad(bases2, ((0, 0), (0, 0), (0, CP - C)))
    root2p = jnp.pad(root2, ((0, 0), (0, CP - C)))
    bias2p = jnp.pad(bias2, (0, CP - C)).reshape(1, CP)

    hall1 = _tc_prep1(xp, comp1, bases1).reshape(R * NP, H)
    a_parts, inv = _sc_msg1(hall1, src, dst, et, z128, z16)
    hall2, self2 = _tc_mid(xp, a_parts, root1, bias1, comp2, bases2p,
                           root2p, bias2p)
    b_parts = _sc_msg2(hall2.reshape(R * NP, CP), src, dst, et, z128, inv)
    out = _tc_final(self2, b_parts)
    return out[:N, :C]
